# Initial kernel scaffold; baseline (speedup 1.0000x reference)
import jax
import jax.numpy as jnp
from jax.experimental import pallas as pl


def kernel(P_raw, admin_ids, census_totals):
    raise NotImplementedError("write your pallas kernel here")


# speedup vs baseline: 296.7105x; 296.7105x over previous
import functools

import jax
import jax.numpy as jnp
from jax import lax
from jax.experimental import pallas as pl
from jax.experimental.pallas import tpu as pltpu
from jax.experimental.pallas import tpu_sc as plsc

EPS_ = 1e-06
B_ = 8
HW_ = 512 * 512
M_ = 1000
MP_ = 1024
NC_ = 2
NS_ = 16
LANES_ = 16
PER_W_ = B_ * HW_ // (NC_ * NS_)
CHUNK_ = 16384
N_CHUNK_ = PER_W_ // CHUNK_


def _sc_body(p_hbm, ids_hbm, census_hbm, out_hbm,
             hist, ratio, crow, quad, idv, pv, ov, shared):
    c = lax.axis_index("c")
    s = lax.axis_index("s")
    batch = c * 4 + s // 4
    base = batch * HW_ + (s % 4) * PER_W_

    zeros16 = jnp.zeros((LANES_,), jnp.float32)

    def zinit(i, carry):
        hist[pl.ds(i * 16, 16)] = zeros16
        return carry

    lax.fori_loop(0, LANES_ * MP_ // 16, zinit, 0)

    lane_off = lax.iota(jnp.int32, 16) * MP_

    def chunk_hist(k, carry):
        off = base + k * CHUNK_
        pltpu.sync_copy(ids_hbm.at[pl.ds(off, CHUNK_)], idv)
        pltpu.sync_copy(p_hbm.at[pl.ds(off, CHUNK_)], pv)

        def inner(i, carry2):
            sl = pl.ds(i * 16, 16)
            plsc.addupdate_scatter(hist, [lane_off + idv[sl]], pv[sl])
            return carry2

        lax.fori_loop(0, CHUNK_ // 16, inner, 0)
        return carry

    lax.fori_loop(0, N_CHUNK_, chunk_hist, 0)

    def lane_red(j, carry):
        acc = hist[pl.ds(j * 16, 16)]
        for l in range(1, LANES_):
            acc = acc + hist[pl.ds(l * MP_ + j * 16, 16)]
        ratio[pl.ds(j * 16, 16)] = acc
        return carry

    lax.fori_loop(0, MP_ // 16, lane_red, 0)

    pltpu.sync_copy(ratio, shared.at[s])
    plsc.subcore_barrier()

    gb = (s // 4) * 4
    pltpu.sync_copy(shared.at[pl.ds(gb, 4)], quad)
    pltpu.sync_copy(census_hbm.at[pl.ds(batch * MP_, MP_)], crow)

    def combine(j, carry):
        sl = pl.ds(j * 16, 16)
        stot = quad[0, sl] + quad[1, sl] + quad[2, sl] + quad[3, sl] + EPS_
        ratio[sl] = crow[sl] / stot
        return carry

    lax.fori_loop(0, MP_ // 16, combine, 0)

    def chunk_norm(k, carry):
        off = base + k * CHUNK_
        pltpu.sync_copy(ids_hbm.at[pl.ds(off, CHUNK_)], idv)
        pltpu.sync_copy(p_hbm.at[pl.ds(off, CHUNK_)], pv)

        def inner(i, carry2):
            sl = pl.ds(i * 16, 16)
            r = plsc.load_gather(ratio, [idv[sl]])
            ov[sl] = pv[sl] * r
            return carry2

        lax.fori_loop(0, CHUNK_ // 16, inner, 0)
        pltpu.sync_copy(ov, out_hbm.at[pl.ds(off, CHUNK_)])
        return carry

    lax.fori_loop(0, N_CHUNK_, chunk_norm, 0)


@jax.jit
def _census_sc(p, ids, census_pad):
    mesh = plsc.VectorSubcoreMesh(core_axis_name="c", subcore_axis_name="s")
    kfn = functools.partial(
        pl.kernel, mesh=mesh,
        out_type=jax.ShapeDtypeStruct((B_ * HW_,), jnp.float32),
        scratch_types=[
            pltpu.VMEM((LANES_ * MP_,), jnp.float32),
            pltpu.VMEM((MP_,), jnp.float32),
            pltpu.VMEM((MP_,), jnp.float32),
            pltpu.VMEM((4, MP_), jnp.float32),
            pltpu.VMEM((CHUNK_,), jnp.int32),
            pltpu.VMEM((CHUNK_,), jnp.float32),
            pltpu.VMEM((CHUNK_,), jnp.float32),
            pltpu.VMEM_SHARED((NS_, MP_), jnp.float32),
        ],
        compiler_params=pltpu.CompilerParams(needs_layout_passes=False),
    )(_sc_body)
    return kfn(p, ids, census_pad)


def kernel(P_raw, admin_ids, census_totals):
    B, C, H, W = P_raw.shape
    p = P_raw.reshape(B * H * W)
    ids = admin_ids.reshape(B * H * W)
    census_pad = jnp.zeros((B, MP_), jnp.float32).at[:, :M_].set(
        census_totals).reshape(-1)
    out = _census_sc(p, ids, census_pad)
    return out.reshape(B, 1, H, W)


# speedup vs baseline: 390.5937x; 1.3164x over previous
import functools

import jax
import jax.numpy as jnp
from jax import lax
from jax.experimental import pallas as pl
from jax.experimental.pallas import tpu as pltpu
from jax.experimental.pallas import tpu_sc as plsc

EPS_ = 1e-06
B_ = 8
H_ = 512
W_ = 512
HW_ = H_ * W_
M_ = 1000
MP_ = 1024
NC_ = 2
NS_ = 16
LANES_ = 16
PER_W_ = B_ * HW_ // (NC_ * NS_)
ROWS_ = 32
CHUNK_ = ROWS_ * W_
N_CHUNK_ = PER_W_ // CHUNK_
ROWS_PER_W_ = PER_W_ // W_


def _sc_body(p_hbm, ids_hbm, census_hbm, out_hbm,
             hist, ratio, crow, quad, idv, pv, ov, shared):
    c = lax.axis_index("c")
    s = lax.axis_index("s")
    batch = c * 4 + s // 4
    row0 = (s % 4) * ROWS_PER_W_

    zeros16 = jnp.zeros((LANES_,), jnp.float32)

    def zinit(i, carry):
        hist[pl.ds(i * 16, 16)] = zeros16
        return carry

    lax.fori_loop(0, LANES_ * MP_ // 16, zinit, 0)

    lane_off = lax.iota(jnp.int32, 16) * MP_

    def chunk_hist(k, carry):
        r = row0 + k * ROWS_
        pltpu.sync_copy(ids_hbm.at[batch, pl.ds(r, ROWS_), :], idv)
        pltpu.sync_copy(p_hbm.at[batch, 0, pl.ds(r, ROWS_), :], pv)

        def inner(i, carry2):
            rr = i >> 5
            cb = (i & 31) * 16
            plsc.addupdate_scatter(
                hist, [lane_off + idv[rr, pl.ds(cb, 16)]], pv[rr, pl.ds(cb, 16)])
            return carry2

        lax.fori_loop(0, CHUNK_ // 16, inner, 0)
        return carry

    lax.fori_loop(0, N_CHUNK_, chunk_hist, 0)

    def lane_red(j, carry):
        acc = hist[pl.ds(j * 16, 16)]
        for l in range(1, LANES_):
            acc = acc + hist[pl.ds(l * MP_ + j * 16, 16)]
        ratio[pl.ds(j * 16, 16)] = acc
        return carry

    lax.fori_loop(0, MP_ // 16, lane_red, 0)

    pltpu.sync_copy(ratio, shared.at[s])
    plsc.subcore_barrier()

    gb = (s // 4) * 4
    pltpu.sync_copy(shared.at[pl.ds(gb, 4)], quad)
    pltpu.sync_copy(census_hbm.at[batch], crow)

    def combine(j, carry):
        sl = pl.ds(j * 16, 16)
        stot = quad[0, sl] + quad[1, sl] + quad[2, sl] + quad[3, sl] + EPS_
        ratio[sl] = crow[sl] / stot
        return carry

    lax.fori_loop(0, MP_ // 16, combine, 0)

    def chunk_norm(k, carry):
        r = row0 + k * ROWS_
        pltpu.sync_copy(ids_hbm.at[batch, pl.ds(r, ROWS_), :], idv)
        pltpu.sync_copy(p_hbm.at[batch, 0, pl.ds(r, ROWS_), :], pv)

        def inner(i, carry2):
            rr = i >> 5
            cb = (i & 31) * 16
            rat = plsc.load_gather(ratio, [idv[rr, pl.ds(cb, 16)]])
            ov[rr, pl.ds(cb, 16)] = pv[rr, pl.ds(cb, 16)] * rat
            return carry2

        lax.fori_loop(0, CHUNK_ // 16, inner, 0)
        pltpu.sync_copy(ov, out_hbm.at[batch, 0, pl.ds(r, ROWS_), :])
        return carry

    lax.fori_loop(0, N_CHUNK_, chunk_norm, 0)


@jax.jit
def _census_sc(P_raw, ids, census_pad):
    mesh = plsc.VectorSubcoreMesh(core_axis_name="c", subcore_axis_name="s")
    kfn = functools.partial(
        pl.kernel, mesh=mesh,
        out_type=jax.ShapeDtypeStruct((B_, 1, H_, W_), jnp.float32),
        scratch_types=[
            pltpu.VMEM((LANES_ * MP_,), jnp.float32),
            pltpu.VMEM((MP_,), jnp.float32),
            pltpu.VMEM((MP_,), jnp.float32),
            pltpu.VMEM((4, MP_), jnp.float32),
            pltpu.VMEM((ROWS_, W_), jnp.int32),
            pltpu.VMEM((ROWS_, W_), jnp.float32),
            pltpu.VMEM((ROWS_, W_), jnp.float32),
            pltpu.VMEM_SHARED((NS_, MP_), jnp.float32),
        ],
        compiler_params=pltpu.CompilerParams(needs_layout_passes=False),
    )(_sc_body)
    return kfn(P_raw, ids, census_pad)


def kernel(P_raw, admin_ids, census_totals):
    census_pad = jnp.zeros((B_, MP_), jnp.float32).at[:, :M_].set(
        census_totals)
    return _census_sc(P_raw, admin_ids, census_pad)


# speedup vs baseline: 874.1430x; 2.2380x over previous
import functools

import jax
import jax.numpy as jnp
from jax import lax
from jax.experimental import pallas as pl
from jax.experimental.pallas import tpu as pltpu
from jax.experimental.pallas import tpu_sc as plsc

EPS_ = 1e-06
B_ = 8
H_ = 512
W_ = 512
HW_ = H_ * W_
M_ = 1000
MP_ = 1024
NC_ = 2
NS_ = 16
LANES_ = 16
PER_W_ = B_ * HW_ // (NC_ * NS_)
ROWS_ = 16
CHUNK_ = ROWS_ * W_
N_CHUNK_ = PER_W_ // CHUNK_
ROWS_PER_W_ = PER_W_ // W_
ITERS_ = CHUNK_ // 16


def _sc_body(p_hbm, ids_hbm, census_hbm, out_hbm,
             hist, ratio, crow, quad,
             idv0, idv1, pv0, pv1, ov0, ov1,
             si0, si1, sp0, sp1, so0, so1, shared):
    c = lax.axis_index("c")
    s = lax.axis_index("s")
    batch = c * 4 + s // 4
    row0 = (s % 4) * ROWS_PER_W_

    idv = (idv0, idv1)
    pv = (pv0, pv1)
    ov = (ov0, ov1)
    si = (si0, si1)
    sp = (sp0, sp1)
    so = (so0, so1)

    def start_in(k):
        bi = k % 2
        r = row0 + k * ROWS_
        c1 = pltpu.async_copy(ids_hbm.at[batch, pl.ds(r, ROWS_), :],
                              idv[bi], si[bi])
        c2 = pltpu.async_copy(p_hbm.at[batch, 0, pl.ds(r, ROWS_), :],
                              pv[bi], sp[bi])
        return (c1, c2)

    cps = [start_in(0), None]

    zeros16 = jnp.zeros((LANES_,), jnp.float32)

    @plsc.parallel_loop(0, LANES_ * MP_ // 16, unroll=8)
    def _(i):
        hist[pl.ds(i * 16, 16)] = zeros16

    lane_off = lax.iota(jnp.int32, 16) * MP_

    for k in range(N_CHUNK_):
        bi = k % 2
        if k + 1 < N_CHUNK_:
            cps[(k + 1) % 2] = start_in(k + 1)
        for d in cps[bi]:
            d.wait()
        ib, vb = idv[bi], pv[bi]

        @plsc.parallel_loop(0, ITERS_, unroll=8)
        def _(i):
            rr = i >> 5
            cb = (i & 31) * 16
            plsc.addupdate_scatter(
                hist, [lane_off + ib[rr, pl.ds(cb, 16)]], vb[rr, pl.ds(cb, 16)])

    @plsc.parallel_loop(0, MP_ // 16, unroll=2)
    def _(j):
        acc = hist[pl.ds(j * 16, 16)]
        for l in range(1, LANES_):
            acc = acc + hist[pl.ds(l * MP_ + j * 16, 16)]
        ratio[pl.ds(j * 16, 16)] = acc

    pltpu.sync_copy(ratio, shared.at[s])

    cps = [start_in(0), start_in(1)]
    ccp = pltpu.async_copy(census_hbm.at[batch], crow, so0)

    plsc.subcore_barrier()

    gb = (s // 4) * 4
    pltpu.sync_copy(shared.at[pl.ds(gb, 4)], quad)
    ccp.wait()

    @plsc.parallel_loop(0, MP_ // 16, unroll=2)
    def _(j):
        sl = pl.ds(j * 16, 16)
        stot = quad[0, sl] + quad[1, sl] + quad[2, sl] + quad[3, sl] + EPS_
        ratio[sl] = crow[sl] / stot

    ocp = [None, None]
    for k in range(N_CHUNK_):
        bi = k % 2
        if k + 1 < N_CHUNK_ and k > 0:
            cps[(k + 1) % 2] = start_in(k + 1)
        for d in cps[bi]:
            d.wait()
        if ocp[bi] is not None:
            ocp[bi].wait()
        ib, vb, ob = idv[bi], pv[bi], ov[bi]

        @plsc.parallel_loop(0, ITERS_, unroll=8)
        def _(i):
            rr = i >> 5
            cb = (i & 31) * 16
            rat = plsc.load_gather(ratio, [ib[rr, pl.ds(cb, 16)]])
            ob[rr, pl.ds(cb, 16)] = vb[rr, pl.ds(cb, 16)] * rat

        r = row0 + k * ROWS_
        ocp[bi] = pltpu.async_copy(
            ov[bi], out_hbm.at[batch, 0, pl.ds(r, ROWS_), :], so[bi])

    ocp[0].wait()
    ocp[1].wait()


@jax.jit
def _census_sc(P_raw, ids, census_pad):
    mesh = plsc.VectorSubcoreMesh(core_axis_name="c", subcore_axis_name="s")
    kfn = functools.partial(
        pl.kernel, mesh=mesh,
        out_type=jax.ShapeDtypeStruct((B_, 1, H_, W_), jnp.float32),
        scratch_types=[
            pltpu.VMEM((LANES_ * MP_,), jnp.float32),
            pltpu.VMEM((MP_,), jnp.float32),
            pltpu.VMEM((MP_,), jnp.float32),
            pltpu.VMEM((4, MP_), jnp.float32),
            pltpu.VMEM((ROWS_, W_), jnp.int32),
            pltpu.VMEM((ROWS_, W_), jnp.int32),
            pltpu.VMEM((ROWS_, W_), jnp.float32),
            pltpu.VMEM((ROWS_, W_), jnp.float32),
            pltpu.VMEM((ROWS_, W_), jnp.float32),
            pltpu.VMEM((ROWS_, W_), jnp.float32),
            pltpu.SemaphoreType.DMA,
            pltpu.SemaphoreType.DMA,
            pltpu.SemaphoreType.DMA,
            pltpu.SemaphoreType.DMA,
            pltpu.SemaphoreType.DMA,
            pltpu.SemaphoreType.DMA,
            pltpu.VMEM_SHARED((NS_, MP_), jnp.float32),
        ],
        compiler_params=pltpu.CompilerParams(needs_layout_passes=False),
    )(_sc_body)
    return kfn(P_raw, ids, census_pad)


def kernel(P_raw, admin_ids, census_totals):
    census_pad = jnp.zeros((B_, MP_), jnp.float32).at[:, :M_].set(
        census_totals)
    return _census_sc(P_raw, admin_ids, census_pad)
